# Initial kernel scaffold; baseline (speedup 1.0000x reference)
import jax
import jax.numpy as jnp
from jax.experimental import pallas as pl


def kernel(graph, features, W1, al1, ar1, b1, W2, al2, ar2, b2, Wres2):
    raise NotImplementedError("write your pallas kernel here")


# speedup vs baseline: 6.3136x; 6.3136x over previous
import functools

import jax
import jax.numpy as jnp
from jax import lax
from jax.experimental import pallas as pl
from jax.experimental.pallas import tpu as pltpu
from jax.experimental.pallas import tpu_sc as plsc

N = 10000
E = 320000
IN = 128
HID = 128
HEADS = 8
OUT = 128

NC = 2
NS = 16
NW = NC * NS
B = 128
NBT = E // B
NP = 10240
RPS = NP // NS
RC = 128


def _sc_edge_pass(nh):
    mesh = plsc.VectorSubcoreMesh(core_axis_name="c", subcore_axis_name="s",
                                  num_cores=NC, num_subcores=NS)

    @functools.partial(
        pl.kernel,
        out_type=(
            jax.ShapeDtypeStruct((NC * nh * NP, HID), jnp.float32),
            jax.ShapeDtypeStruct((NW * nh * NP,), jnp.float32),
        ),
        mesh=mesh,
        scratch_types=[
            pltpu.VMEM_SHARED((NP, HID), jnp.float32),
            pltpu.VMEM((NP,), jnp.float32),
            pltpu.VMEM((NP,), jnp.float32),
            pltpu.VMEM((NP,), jnp.float32),
            pltpu.VMEM((B, HID), jnp.float32),
            pltpu.VMEM((B,), jnp.int32),
            pltpu.VMEM((B,), jnp.int32),
            pltpu.VMEM((B,), jnp.int32),
            pltpu.VMEM((B,), jnp.float32),
        ],
        compiler_params=pltpu.CompilerParams(needs_layout_passes=False),
    )
    def body(feat_hbm, el_hbm, er_hbm, src_hbm, dst_hbm,
             acc_hbm, den_hbm,
             accum_sh, el_v, er_v, den_v, rows_v,
             src_b, dst_b, sadj_b, s_b):
        c = lax.axis_index("c")
        s = lax.axis_index("s")
        wid = c * NS + s
        nb = jnp.where(wid < NBT - (NBT // NW) * NW, NBT // NW + 1, NBT // NW)

        def head_step(h, _):
            pltpu.sync_copy(el_hbm.at[pl.ds(h * NP, NP)], el_v)
            pltpu.sync_copy(er_hbm.at[pl.ds(h * NP, NP)], er_v)

            def zden(i, _):
                den_v[pl.ds(i * 16, 16)] = jnp.zeros((16,), jnp.float32)
                return _

            lax.fori_loop(0, NP // 16, zden, 0)

            def zrows(j, _):
                for i in range(HID // 16):
                    rows_v[j, pl.ds(i * 16, 16)] = jnp.zeros((16,), jnp.float32)
                return _

            lax.fori_loop(0, B, zrows, 0)
            for i in range(RPS // RC):
                pltpu.sync_copy(rows_v, accum_sh.at[pl.ds(s * RPS + i * RC, RC)])
            plsc.subcore_barrier()

            def batch_step(t, _):
                base = (wid + NW * t) * B
                pltpu.sync_copy(src_hbm.at[pl.ds(base, B)], src_b)
                pltpu.sync_copy(dst_hbm.at[pl.ds(base, B)], dst_b)
                for k in range(B // 16):
                    sv = src_b[pl.ds(k * 16, 16)]
                    dv = dst_b[pl.ds(k * 16, 16)]
                    sadj_b[pl.ds(k * 16, 16)] = sv + h * N
                    e = (plsc.load_gather(el_v, [sv])
                         + plsc.load_gather(er_v, [dv]))
                    e = jnp.where(e < 0, e * jnp.float32(0.2), e)
                    sval = jnp.exp(e)
                    s_b[pl.ds(k * 16, 16)] = sval
                    plsc.addupdate_scatter(den_v, [dv], sval)
                pltpu.sync_copy(feat_hbm.at[sadj_b], rows_v)

                def row_step(j, _):
                    sj = plsc.load_gather(s_b, [jnp.full((16,), j, jnp.int32)])
                    for i in range(HID // 16):
                        rows_v[j, pl.ds(i * 16, 16)] = (
                            rows_v[j, pl.ds(i * 16, 16)] * sj)
                    return _

                lax.fori_loop(0, B, row_step, 0)
                pltpu.sync_copy(rows_v, accum_sh.at[dst_b], add=True)
                return _

            lax.fori_loop(0, nb, batch_step, 0)
            plsc.subcore_barrier()
            for i in range(RPS // RC):
                r = s * RPS + i * RC
                pltpu.sync_copy(
                    accum_sh.at[pl.ds(r, RC)],
                    acc_hbm.at[pl.ds((c * nh + h) * NP + r, RC)])
            pltpu.sync_copy(den_v, den_hbm.at[pl.ds((wid * nh + h) * NP, NP)])
            plsc.subcore_barrier()
            return _

        lax.fori_loop(0, nh, head_step, 0)

    return body


_sc_pass_l1 = _sc_edge_pass(HEADS)
_sc_pass_l2 = _sc_edge_pass(1)

R = 200
GRID = N // R


def _prep1_body(x_ref, w1_ref, al_ref, ar_ref, feat_ref, el_ref, er_ref):
    x = x_ref[...]
    els, ers = [], []
    for h in range(HEADS):
        fh = jnp.dot(x, w1_ref[:, pl.ds(h * HID, HID)],
                     preferred_element_type=jnp.float32)
        feat_ref[h] = fh
        els.append(jnp.sum(fh * al_ref[h, :][None, :], axis=1))
        ers.append(jnp.sum(fh * ar_ref[h, :][None, :], axis=1))
    el_ref[...] = jnp.stack(els, axis=1)
    er_ref[...] = jnp.stack(ers, axis=1)


def _prep1(features, W1, al1, ar1):
    return pl.pallas_call(
        _prep1_body,
        grid=(GRID,),
        in_specs=[
            pl.BlockSpec((R, IN), lambda i: (i, 0)),
            pl.BlockSpec((IN, HEADS * HID), lambda i: (0, 0)),
            pl.BlockSpec((HEADS, HID), lambda i: (0, 0)),
            pl.BlockSpec((HEADS, HID), lambda i: (0, 0)),
        ],
        out_specs=[
            pl.BlockSpec((HEADS, R, HID), lambda i: (0, i, 0)),
            pl.BlockSpec((R, HEADS), lambda i: (i, 0)),
            pl.BlockSpec((R, HEADS), lambda i: (i, 0)),
        ],
        out_shape=[
            jax.ShapeDtypeStruct((HEADS, N, HID), jnp.float32),
            jax.ShapeDtypeStruct((N, HEADS), jnp.float32),
            jax.ShapeDtypeStruct((N, HEADS), jnp.float32),
        ],
    )(features, W1, al1, ar1)


def _dense2_body(acc_ref, den_ref, b1_ref, w2_ref, wres_ref, al_ref, ar_ref,
                 feat2_ref, el_ref, er_ref, res_ref):
    a = acc_ref[...]
    comb = a[0] + a[1]
    dsum = jnp.sum(den_ref[...], axis=2)
    f2 = jnp.zeros((R, OUT), jnp.float32)
    r2 = jnp.zeros((R, OUT), jnp.float32)
    for h in range(HEADS):
        den = dsum[:, h]
        num = comb[h]
        safe = jnp.where(den > 0, den, jnp.float32(1.0))[:, None]
        hh = jnp.where(den[:, None] > 0, num / safe, jnp.float32(0.0))
        hh = hh + b1_ref[h, :][None, :]
        hh = jnp.where(hh > 0, hh, jnp.exp(jnp.minimum(hh, 0.0)) - 1.0)
        f2 = f2 + jnp.dot(hh, w2_ref[pl.ds(h * HID, HID), :],
                          preferred_element_type=jnp.float32)
        r2 = r2 + jnp.dot(hh, wres_ref[pl.ds(h * HID, HID), :],
                          preferred_element_type=jnp.float32)
    feat2_ref[...] = f2
    el_ref[...] = jnp.sum(f2 * al_ref[0, :][None, :], axis=1)[:, None]
    er_ref[...] = jnp.sum(f2 * ar_ref[0, :][None, :], axis=1)[:, None]
    res_ref[...] = r2


def _dense2(acc1, den1, b1, W2, Wres2, al2, ar2):
    return pl.pallas_call(
        _dense2_body,
        grid=(GRID,),
        in_specs=[
            pl.BlockSpec((NC, HEADS, R, HID), lambda i: (0, 0, i, 0)),
            pl.BlockSpec((R, HEADS, NW), lambda i: (i, 0, 0)),
            pl.BlockSpec((HEADS, HID), lambda i: (0, 0)),
            pl.BlockSpec((HEADS * HID, OUT), lambda i: (0, 0)),
            pl.BlockSpec((HEADS * HID, OUT), lambda i: (0, 0)),
            pl.BlockSpec((1, OUT), lambda i: (0, 0)),
            pl.BlockSpec((1, OUT), lambda i: (0, 0)),
        ],
        out_specs=[
            pl.BlockSpec((R, HID), lambda i: (i, 0)),
            pl.BlockSpec((R, 1), lambda i: (i, 0)),
            pl.BlockSpec((R, 1), lambda i: (i, 0)),
            pl.BlockSpec((R, OUT), lambda i: (i, 0)),
        ],
        out_shape=[
            jax.ShapeDtypeStruct((N, HID), jnp.float32),
            jax.ShapeDtypeStruct((NP, 1), jnp.float32),
            jax.ShapeDtypeStruct((NP, 1), jnp.float32),
            jax.ShapeDtypeStruct((N, OUT), jnp.float32),
        ],
    )(acc1, den1, b1, W2, Wres2, al2, ar2)


def _final_body(acc_ref, den_ref, res_ref, b2_ref, out_ref):
    a = acc_ref[...]
    comb = a[0, 0] + a[1, 0]
    den = jnp.sum(den_ref[...], axis=2)[:, 0]
    safe = jnp.where(den > 0, den, jnp.float32(1.0))[:, None]
    rst = jnp.where(den[:, None] > 0, comb / safe, jnp.float32(0.0))
    out_ref[...] = rst + res_ref[...] + b2_ref[0, :][None, :]


def _final(acc2, den2, res, b2):
    return pl.pallas_call(
        _final_body,
        grid=(GRID,),
        in_specs=[
            pl.BlockSpec((NC, 1, R, HID), lambda i: (0, 0, i, 0)),
            pl.BlockSpec((R, 1, NW), lambda i: (i, 0, 0)),
            pl.BlockSpec((R, OUT), lambda i: (i, 0)),
            pl.BlockSpec((1, OUT), lambda i: (0, 0)),
        ],
        out_specs=pl.BlockSpec((R, OUT), lambda i: (i, 0)),
        out_shape=jax.ShapeDtypeStruct((N, OUT), jnp.float32),
    )(acc2, den2, res, b2)


def _pad_nodes(x):
    k = x.shape[1]
    return jnp.pad(x.T, ((0, 0), (0, NP - N))).reshape(k * NP)


def kernel(graph, features, W1, al1, ar1, b1, W2, al2, ar2, b2, Wres2):
    src = graph[0].astype(jnp.int32)
    dst = graph[1].astype(jnp.int32)

    feat1, el1, er1 = _prep1(features, W1, al1, ar1)
    acc1, den1 = _sc_pass_l1(feat1.reshape(HEADS * N, HID),
                             _pad_nodes(el1), _pad_nodes(er1), src, dst)
    den1_t = den1.reshape(NW, HEADS, NP).transpose(2, 1, 0)
    feat2, el2, er2, res = _dense2(acc1.reshape(NC, HEADS, NP, HID),
                                   den1_t, b1, W2, Wres2, al2, ar2)
    acc2, den2 = _sc_pass_l2(feat2, el2.reshape(-1), er2.reshape(-1),
                             src, dst)
    den2_t = den2.reshape(NW, 1, NP).transpose(2, 1, 0)
    return _final(acc2.reshape(NC, 1, NP, HID), den2_t, res, b2)


# speedup vs baseline: 16.8271x; 2.6652x over previous
import functools

import jax
import jax.numpy as jnp
from jax import lax
from jax.experimental import pallas as pl
from jax.experimental.pallas import tpu as pltpu
from jax.experimental.pallas import tpu_sc as plsc

N = 10000
E = 320000
IN = 128
HID = 128
HEADS = 8
OUT = 128

NC = 2
NS = 16
NW = NC * NS
B = 128
CB = 4
CE = CB * B
NCH = E // CE
NP = 10240
RPS = NP // NS
RC = 128
HPC = HEADS // NC


def _sc_edge_pass(nh, head_split):
    ncopies = 1 if head_split else NC
    mesh = plsc.VectorSubcoreMesh(core_axis_name="c", subcore_axis_name="s",
                                  num_cores=NC, num_subcores=NS)

    @functools.partial(
        pl.kernel,
        out_type=(
            jax.ShapeDtypeStruct((ncopies * nh * NP, HID), jnp.float32),
            jax.ShapeDtypeStruct((ncopies * nh * NP,), jnp.float32),
        ),
        mesh=mesh,
        scratch_types=[
            pltpu.VMEM_SHARED((NP, HID), jnp.float32),
            pltpu.VMEM_SHARED((NP,), jnp.float32),
            pltpu.VMEM((NP,), jnp.float32),
            pltpu.VMEM((NP,), jnp.float32),
            pltpu.VMEM((B, HID), jnp.float32),
            pltpu.VMEM((CE,), jnp.int32),
            pltpu.VMEM((CE,), jnp.int32),
            pltpu.VMEM((B,), jnp.int32),
            pltpu.VMEM((B,), jnp.int32),
            pltpu.VMEM((B,), jnp.float32),
        ],
        compiler_params=pltpu.CompilerParams(needs_layout_passes=False),
    )
    def body(feat_hbm, el_hbm, er_hbm, src_hbm, dst_hbm,
             acc_hbm, den_hbm,
             accum_sh, den_sh, el_v, er_v, rows_v,
             src_c, dst_c, dst_b, sadj_b, s_b):
        c = lax.axis_index("c")
        s = lax.axis_index("s")
        if head_split:
            lane, stride = s, NS
        else:
            lane, stride = c * NS + s, NW
        nq = jnp.where(lane < NCH - (NCH // stride) * stride,
                       NCH // stride + 1, NCH // stride)
        nheads = HPC if head_split else nh

        def head_step(hl, _):
            h = c * HPC + hl if head_split else hl
            pltpu.sync_copy(el_hbm.at[pl.ds(h * NP, NP)], el_v)
            pltpu.sync_copy(er_hbm.at[pl.ds(h * NP, NP)], er_v)

            for k in range(B // 16):
                s_b[pl.ds(k * 16, 16)] = jnp.zeros((16,), jnp.float32)

            def zrows(j, _):
                for i in range(HID // 16):
                    rows_v[j, pl.ds(i * 16, 16)] = jnp.zeros((16,), jnp.float32)
                return _

            lax.fori_loop(0, B, zrows, 0)
            for i in range(RPS // RC):
                pltpu.sync_copy(rows_v, accum_sh.at[pl.ds(s * RPS + i * RC, RC)])
                pltpu.sync_copy(s_b, den_sh.at[pl.ds(s * RPS + i * RC, RC)])
            plsc.subcore_barrier()

            def chunk_step(t, _):
                base = (lane + stride * t) * CE
                pltpu.sync_copy(src_hbm.at[pl.ds(base, CE)], src_c)
                pltpu.sync_copy(dst_hbm.at[pl.ds(base, CE)], dst_c)
                for kb in range(CB):
                    for k in range(B // 16):
                        o = kb * B + k * 16
                        sv = src_c[pl.ds(o, 16)]
                        dv = dst_c[pl.ds(o, 16)]
                        sadj_b[pl.ds(k * 16, 16)] = sv + h * N
                        dst_b[pl.ds(k * 16, 16)] = dv
                        e = (plsc.load_gather(el_v, [sv])
                             + plsc.load_gather(er_v, [dv]))
                        e = jnp.where(e < 0, e * jnp.float32(0.2), e)
                        sval = jnp.exp(e)
                        s_b[pl.ds(k * 16, 16)] = sval
                    pltpu.sync_copy(s_b, den_sh.at[dst_b], add=True)
                    pltpu.sync_copy(feat_hbm.at[sadj_b], rows_v)

                    def row_step(j, _):
                        sj = plsc.load_gather(
                            s_b, [jnp.full((16,), j, jnp.int32)])
                        for i in range(HID // 16):
                            rows_v[j, pl.ds(i * 16, 16)] = (
                                rows_v[j, pl.ds(i * 16, 16)] * sj)
                        return _

                    lax.fori_loop(0, B, row_step, 0)
                    pltpu.sync_copy(rows_v, accum_sh.at[dst_b], add=True)
                return _

            lax.fori_loop(0, nq, chunk_step, 0)
            plsc.subcore_barrier()
            off = (h if head_split else c * nh + h) * NP
            for i in range(RPS // RC):
                r = s * RPS + i * RC
                pltpu.sync_copy(accum_sh.at[pl.ds(r, RC)],
                                acc_hbm.at[pl.ds(off + r, RC)])
            pltpu.sync_copy(den_sh.at[pl.ds(s * RPS, RPS)],
                            den_hbm.at[pl.ds(off + s * RPS, RPS)])
            plsc.subcore_barrier()
            return _

        lax.fori_loop(0, nheads, head_step, 0)

    return body


_sc_pass_l1 = _sc_edge_pass(HEADS, True)
_sc_pass_l2 = _sc_edge_pass(1, False)

R = 200
GRID = N // R


def _prep1_body(x_ref, w1_ref, al_ref, ar_ref, feat_ref, el_ref, er_ref):
    x = x_ref[...]
    els, ers = [], []
    for h in range(HEADS):
        fh = jnp.dot(x, w1_ref[:, pl.ds(h * HID, HID)],
                     preferred_element_type=jnp.float32)
        feat_ref[h] = fh
        els.append(jnp.sum(fh * al_ref[h, :][None, :], axis=1))
        ers.append(jnp.sum(fh * ar_ref[h, :][None, :], axis=1))
    el_ref[...] = jnp.stack(els, axis=1)
    er_ref[...] = jnp.stack(ers, axis=1)


def _prep1(features, W1, al1, ar1):
    return pl.pallas_call(
        _prep1_body,
        grid=(GRID,),
        in_specs=[
            pl.BlockSpec((R, IN), lambda i: (i, 0)),
            pl.BlockSpec((IN, HEADS * HID), lambda i: (0, 0)),
            pl.BlockSpec((HEADS, HID), lambda i: (0, 0)),
            pl.BlockSpec((HEADS, HID), lambda i: (0, 0)),
        ],
        out_specs=[
            pl.BlockSpec((HEADS, R, HID), lambda i: (0, i, 0)),
            pl.BlockSpec((R, HEADS), lambda i: (i, 0)),
            pl.BlockSpec((R, HEADS), lambda i: (i, 0)),
        ],
        out_shape=[
            jax.ShapeDtypeStruct((HEADS, N, HID), jnp.float32),
            jax.ShapeDtypeStruct((N, HEADS), jnp.float32),
            jax.ShapeDtypeStruct((N, HEADS), jnp.float32),
        ],
    )(features, W1, al1, ar1)


def _dense2_body(acc_ref, den_ref, b1_ref, wcat_ref, al_ref, ar_ref,
                 feat2_ref, el_ref, er_ref, res_ref):
    hs = []
    for h in range(HEADS):
        den = den_ref[:, h]
        num = acc_ref[h]
        safe = jnp.where(den > 0, den, jnp.float32(1.0))[:, None]
        hh = jnp.where(den[:, None] > 0, num / safe, jnp.float32(0.0))
        hh = hh + b1_ref[h, :][None, :]
        hh = jnp.where(hh > 0, hh, jnp.exp(jnp.minimum(hh, 0.0)) - 1.0)
        hs.append(hh)
    hcat = jnp.concatenate(hs, axis=1)
    both = jnp.dot(hcat, wcat_ref[...],
                   preferred_element_type=jnp.float32)
    f2 = both[:, :OUT]
    feat2_ref[...] = f2
    el_ref[...] = jnp.sum(f2 * al_ref[0, :][None, :], axis=1)[:, None]
    er_ref[...] = jnp.sum(f2 * ar_ref[0, :][None, :], axis=1)[:, None]
    res_ref[...] = both[:, OUT:]


def _dense2(acc1, den1, b1, Wcat, al2, ar2):
    return pl.pallas_call(
        _dense2_body,
        grid=(GRID,),
        in_specs=[
            pl.BlockSpec((HEADS, R, HID), lambda i: (0, i, 0)),
            pl.BlockSpec((R, HEADS), lambda i: (i, 0)),
            pl.BlockSpec((HEADS, HID), lambda i: (0, 0)),
            pl.BlockSpec((HEADS * HID, 2 * OUT), lambda i: (0, 0)),
            pl.BlockSpec((1, OUT), lambda i: (0, 0)),
            pl.BlockSpec((1, OUT), lambda i: (0, 0)),
        ],
        out_specs=[
            pl.BlockSpec((R, HID), lambda i: (i, 0)),
            pl.BlockSpec((R, 1), lambda i: (i, 0)),
            pl.BlockSpec((R, 1), lambda i: (i, 0)),
            pl.BlockSpec((R, OUT), lambda i: (i, 0)),
        ],
        out_shape=[
            jax.ShapeDtypeStruct((N, HID), jnp.float32),
            jax.ShapeDtypeStruct((NP, 1), jnp.float32),
            jax.ShapeDtypeStruct((NP, 1), jnp.float32),
            jax.ShapeDtypeStruct((N, OUT), jnp.float32),
        ],
    )(acc1, den1, b1, Wcat, al2, ar2)


def _final_body(acc_ref, den_ref, res_ref, b2_ref, out_ref):
    comb = acc_ref[0] + acc_ref[1]
    den = den_ref[:, 0] + den_ref[:, 1]
    safe = jnp.where(den > 0, den, jnp.float32(1.0))[:, None]
    rst = jnp.where(den[:, None] > 0, comb / safe, jnp.float32(0.0))
    out_ref[...] = rst + res_ref[...] + b2_ref[0, :][None, :]


def _final(acc2, den2, res, b2):
    return pl.pallas_call(
        _final_body,
        grid=(GRID,),
        in_specs=[
            pl.BlockSpec((NC, R, HID), lambda i: (0, i, 0)),
            pl.BlockSpec((R, NC), lambda i: (i, 0)),
            pl.BlockSpec((R, OUT), lambda i: (i, 0)),
            pl.BlockSpec((1, OUT), lambda i: (0, 0)),
        ],
        out_specs=pl.BlockSpec((R, OUT), lambda i: (i, 0)),
        out_shape=jax.ShapeDtypeStruct((N, OUT), jnp.float32),
    )(acc2, den2, res, b2)


def _pad_nodes(x):
    k = x.shape[1]
    return jnp.pad(x.T, ((0, 0), (0, NP - N))).reshape(k * NP)


def kernel(graph, features, W1, al1, ar1, b1, W2, al2, ar2, b2, Wres2):
    src = graph[0].astype(jnp.int32)
    dst = graph[1].astype(jnp.int32)

    feat1, el1, er1 = _prep1(features, W1, al1, ar1)
    acc1, den1 = _sc_pass_l1(feat1.reshape(HEADS * N, HID),
                             _pad_nodes(el1), _pad_nodes(er1), src, dst)
    Wcat = jnp.concatenate([W2, Wres2], axis=1)
    feat2, el2, er2, res = _dense2(acc1.reshape(HEADS, NP, HID),
                                   den1.reshape(HEADS, NP).T,
                                   b1, Wcat, al2, ar2)
    acc2, den2 = _sc_pass_l2(feat2, el2.reshape(-1), er2.reshape(-1),
                             src, dst)
    return _final(acc2.reshape(NC, NP, HID), den2.reshape(NC, NP).T,
                  res, b2)


# speedup vs baseline: 21.6650x; 1.2875x over previous
import functools

import jax
import jax.numpy as jnp
from jax import lax
from jax.experimental import pallas as pl
from jax.experimental.pallas import tpu as pltpu
from jax.experimental.pallas import tpu_sc as plsc

N = 10000
E = 320000
IN = 128
HID = 128
HEADS = 8
OUT = 128

NC = 2
NS = 16
NW = NC * NS
B = 64
CB = 8
CE = CB * B
NCH = E // CE
NP = 10240
RPS = NP // NS
RC = 128
HPC = HEADS // NC


def _sc_edge_pass(nh, head_split):
    ncopies = 1 if head_split else NC
    mesh = plsc.VectorSubcoreMesh(core_axis_name="c", subcore_axis_name="s",
                                  num_cores=NC, num_subcores=NS)

    @functools.partial(
        pl.kernel,
        out_type=(
            jax.ShapeDtypeStruct((ncopies * nh * NP, HID), jnp.float32),
            jax.ShapeDtypeStruct((ncopies * nh * NP,), jnp.float32),
        ),
        mesh=mesh,
        scratch_types=[
            pltpu.VMEM_SHARED((NP, HID), jnp.float32),
            pltpu.VMEM_SHARED((NP,), jnp.float32),
            pltpu.VMEM((NP,), jnp.float32),
            pltpu.VMEM((NP,), jnp.float32),
            pltpu.VMEM((B, HID), jnp.float32),
            pltpu.VMEM((B, HID), jnp.float32),
            pltpu.VMEM((CE,), jnp.int32),
            pltpu.VMEM((CE,), jnp.int32),
            pltpu.VMEM((B,), jnp.int32),
            pltpu.VMEM((B,), jnp.int32),
            pltpu.VMEM((B,), jnp.int32),
            pltpu.VMEM((B,), jnp.int32),
            pltpu.VMEM((B,), jnp.float32),
            pltpu.VMEM((B,), jnp.float32),
            pltpu.SemaphoreType.DMA,
            pltpu.SemaphoreType.DMA,
        ],
        compiler_params=pltpu.CompilerParams(needs_layout_passes=False),
    )
    def body(feat_hbm, el_hbm, er_hbm, src_hbm, dst_hbm,
             acc_hbm, den_hbm,
             accum_sh, den_sh, el_v, er_v, rows_v0, rows_v1,
             src_c, dst_c, dst_b0, dst_b1, sadj_b0, sadj_b1,
             s_b0, s_b1, sem0, sem1):
        bufs = [(rows_v0, dst_b0, sadj_b0, s_b0, sem0),
                (rows_v1, dst_b1, sadj_b1, s_b1, sem1)]
        c = lax.axis_index("c")
        s = lax.axis_index("s")
        if head_split:
            lane, stride = s, NS
        else:
            lane, stride = c * NS + s, NW
        nq = jnp.where(lane < NCH - (NCH // stride) * stride,
                       NCH // stride + 1, NCH // stride)
        nheads = HPC if head_split else nh

        def head_step(hl, _):
            h = c * HPC + hl if head_split else hl
            pltpu.sync_copy(el_hbm.at[pl.ds(h * NP, NP)], el_v)
            pltpu.sync_copy(er_hbm.at[pl.ds(h * NP, NP)], er_v)

            for k in range(B // 16):
                s_b0[pl.ds(k * 16, 16)] = jnp.zeros((16,), jnp.float32)

            def zrows(j, _):
                for i in range(HID // 16):
                    rows_v0[j, pl.ds(i * 16, 16)] = (
                        jnp.zeros((16,), jnp.float32))
                return _

            lax.fori_loop(0, B, zrows, 0)
            for i in range(RPS // B):
                pltpu.sync_copy(rows_v0, accum_sh.at[pl.ds(s * RPS + i * B, B)])
                pltpu.sync_copy(s_b0, den_sh.at[pl.ds(s * RPS + i * B, B)])
            plsc.subcore_barrier()

            def compute(kb, base, p):
                rows_v, dst_b, sadj_b, s_b, sem = bufs[p]
                for k in range(B // 16):
                    o = kb * B + k * 16
                    sv = src_c[pl.ds(o, 16)]
                    dv = dst_c[pl.ds(o, 16)]
                    sadj_b[pl.ds(k * 16, 16)] = sv + h * N
                    dst_b[pl.ds(k * 16, 16)] = dv
                    e = (plsc.load_gather(el_v, [sv])
                         + plsc.load_gather(er_v, [dv]))
                    e = jnp.where(e < 0, e * jnp.float32(0.2), e)
                    s_b[pl.ds(k * 16, 16)] = jnp.exp(e)
                pltpu.sync_copy(s_b, den_sh.at[dst_b], add=True)
                return pltpu.async_copy(feat_hbm.at[sadj_b], rows_v, sem)

            def process(p, handle):
                rows_v, dst_b, sadj_b, s_b, sem = bufs[p]
                handle.wait()

                def row_step(j, _):
                    sj = plsc.load_gather(
                        s_b, [jnp.full((16,), j, jnp.int32)])
                    for i in range(HID // 16):
                        rows_v[j, pl.ds(i * 16, 16)] = (
                            rows_v[j, pl.ds(i * 16, 16)] * sj)
                    return _

                lax.fori_loop(0, B, row_step, 0)
                pltpu.sync_copy(rows_v, accum_sh.at[dst_b], add=True)

            def chunk_step(t, _):
                base = (lane + stride * t) * CE
                pltpu.sync_copy(src_hbm.at[pl.ds(base, CE)], src_c)
                pltpu.sync_copy(dst_hbm.at[pl.ds(base, CE)], dst_c)
                handle = compute(0, base, 0)
                for kb in range(1, CB):
                    nxt = compute(kb, base, kb % 2)
                    process((kb - 1) % 2, handle)
                    handle = nxt
                process((CB - 1) % 2, handle)
                return _

            lax.fori_loop(0, nq, chunk_step, 0)
            plsc.subcore_barrier()
            off = (h if head_split else c * nh + h) * NP
            for i in range(RPS // RC):
                r = s * RPS + i * RC
                pltpu.sync_copy(accum_sh.at[pl.ds(r, RC)],
                                acc_hbm.at[pl.ds(off + r, RC)])
            pltpu.sync_copy(den_sh.at[pl.ds(s * RPS, RPS)],
                            den_hbm.at[pl.ds(off + s * RPS, RPS)])
            plsc.subcore_barrier()
            return _

        lax.fori_loop(0, nheads, head_step, 0)

    return body


_sc_pass_l1 = _sc_edge_pass(HEADS, True)
_sc_pass_l2 = _sc_edge_pass(1, False)

R = 200
GRID = N // R


def _prep1_body(x_ref, w1_ref, al_ref, ar_ref, feat_ref, el_ref, er_ref):
    x = x_ref[...]
    els, ers = [], []
    for h in range(HEADS):
        fh = jnp.dot(x, w1_ref[:, pl.ds(h * HID, HID)],
                     preferred_element_type=jnp.float32)
        feat_ref[h] = fh
        els.append(jnp.sum(fh * al_ref[h, :][None, :], axis=1))
        ers.append(jnp.sum(fh * ar_ref[h, :][None, :], axis=1))
    el_ref[...] = jnp.stack(els, axis=1)
    er_ref[...] = jnp.stack(ers, axis=1)


def _prep1(features, W1, al1, ar1):
    return pl.pallas_call(
        _prep1_body,
        grid=(GRID,),
        in_specs=[
            pl.BlockSpec((R, IN), lambda i: (i, 0)),
            pl.BlockSpec((IN, HEADS * HID), lambda i: (0, 0)),
            pl.BlockSpec((HEADS, HID), lambda i: (0, 0)),
            pl.BlockSpec((HEADS, HID), lambda i: (0, 0)),
        ],
        out_specs=[
            pl.BlockSpec((HEADS, R, HID), lambda i: (0, i, 0)),
            pl.BlockSpec((R, HEADS), lambda i: (i, 0)),
            pl.BlockSpec((R, HEADS), lambda i: (i, 0)),
        ],
        out_shape=[
            jax.ShapeDtypeStruct((HEADS, N, HID), jnp.float32),
            jax.ShapeDtypeStruct((N, HEADS), jnp.float32),
            jax.ShapeDtypeStruct((N, HEADS), jnp.float32),
        ],
    )(features, W1, al1, ar1)


def _dense2_body(acc_ref, den_ref, b1_ref, wcat_ref, al_ref, ar_ref,
                 feat2_ref, el_ref, er_ref, res_ref):
    hs = []
    for h in range(HEADS):
        den = den_ref[:, h]
        num = acc_ref[h]
        safe = jnp.where(den > 0, den, jnp.float32(1.0))[:, None]
        hh = jnp.where(den[:, None] > 0, num / safe, jnp.float32(0.0))
        hh = hh + b1_ref[h, :][None, :]
        hh = jnp.where(hh > 0, hh, jnp.exp(jnp.minimum(hh, 0.0)) - 1.0)
        hs.append(hh)
    hcat = jnp.concatenate(hs, axis=1)
    both = jnp.dot(hcat, wcat_ref[...],
                   preferred_element_type=jnp.float32)
    f2 = both[:, :OUT]
    feat2_ref[...] = f2
    el_ref[...] = jnp.sum(f2 * al_ref[0, :][None, :], axis=1)[:, None]
    er_ref[...] = jnp.sum(f2 * ar_ref[0, :][None, :], axis=1)[:, None]
    res_ref[...] = both[:, OUT:]


def _dense2(acc1, den1, b1, Wcat, al2, ar2):
    return pl.pallas_call(
        _dense2_body,
        grid=(GRID,),
        in_specs=[
            pl.BlockSpec((HEADS, R, HID), lambda i: (0, i, 0)),
            pl.BlockSpec((R, HEADS), lambda i: (i, 0)),
            pl.BlockSpec((HEADS, HID), lambda i: (0, 0)),
            pl.BlockSpec((HEADS * HID, 2 * OUT), lambda i: (0, 0)),
            pl.BlockSpec((1, OUT), lambda i: (0, 0)),
            pl.BlockSpec((1, OUT), lambda i: (0, 0)),
        ],
        out_specs=[
            pl.BlockSpec((R, HID), lambda i: (i, 0)),
            pl.BlockSpec((R, 1), lambda i: (i, 0)),
            pl.BlockSpec((R, 1), lambda i: (i, 0)),
            pl.BlockSpec((R, OUT), lambda i: (i, 0)),
        ],
        out_shape=[
            jax.ShapeDtypeStruct((N, HID), jnp.float32),
            jax.ShapeDtypeStruct((NP, 1), jnp.float32),
            jax.ShapeDtypeStruct((NP, 1), jnp.float32),
            jax.ShapeDtypeStruct((N, OUT), jnp.float32),
        ],
    )(acc1, den1, b1, Wcat, al2, ar2)


def _final_body(acc_ref, den_ref, res_ref, b2_ref, out_ref):
    comb = acc_ref[0] + acc_ref[1]
    den = den_ref[:, 0] + den_ref[:, 1]
    safe = jnp.where(den > 0, den, jnp.float32(1.0))[:, None]
    rst = jnp.where(den[:, None] > 0, comb / safe, jnp.float32(0.0))
    out_ref[...] = rst + res_ref[...] + b2_ref[0, :][None, :]


def _final(acc2, den2, res, b2):
    return pl.pallas_call(
        _final_body,
        grid=(GRID,),
        in_specs=[
            pl.BlockSpec((NC, R, HID), lambda i: (0, i, 0)),
            pl.BlockSpec((R, NC), lambda i: (i, 0)),
            pl.BlockSpec((R, OUT), lambda i: (i, 0)),
            pl.BlockSpec((1, OUT), lambda i: (0, 0)),
        ],
        out_specs=pl.BlockSpec((R, OUT), lambda i: (i, 0)),
        out_shape=jax.ShapeDtypeStruct((N, OUT), jnp.float32),
    )(acc2, den2, res, b2)


def _pad_nodes(x):
    k = x.shape[1]
    return jnp.pad(x.T, ((0, 0), (0, NP - N))).reshape(k * NP)


def kernel(graph, features, W1, al1, ar1, b1, W2, al2, ar2, b2, Wres2):
    src = graph[0].astype(jnp.int32)
    dst = graph[1].astype(jnp.int32)

    feat1, el1, er1 = _prep1(features, W1, al1, ar1)
    acc1, den1 = _sc_pass_l1(feat1.reshape(HEADS * N, HID),
                             _pad_nodes(el1), _pad_nodes(er1), src, dst)
    Wcat = jnp.concatenate([W2, Wres2], axis=1)
    feat2, el2, er2, res = _dense2(acc1.reshape(HEADS, NP, HID),
                                   den1.reshape(HEADS, NP).T,
                                   b1, Wcat, al2, ar2)
    acc2, den2 = _sc_pass_l2(feat2, el2.reshape(-1), er2.reshape(-1),
                             src, dst)
    return _final(acc2.reshape(NC, NP, HID), den2.reshape(NC, NP).T,
                  res, b2)


# speedup vs baseline: 23.3191x; 1.0763x over previous
import functools

import jax
import jax.numpy as jnp
from jax import lax
from jax.experimental import pallas as pl
from jax.experimental.pallas import tpu as pltpu
from jax.experimental.pallas import tpu_sc as plsc

N = 10000
E = 320000
IN = 128
HID = 128
HEADS = 8
OUT = 128

NC = 2
NS = 16
NW = NC * NS
B = 64
CB = 8
CE = CB * B
NCH = E // CE
NP = 10240
RPS = NP // NS
RC = 128
HPC = HEADS // NC


def _sc_edge_pass(nh, head_split):
    ncopies = 1 if head_split else NC
    mesh = plsc.VectorSubcoreMesh(core_axis_name="c", subcore_axis_name="s",
                                  num_cores=NC, num_subcores=NS)

    @functools.partial(
        pl.kernel,
        out_type=(
            jax.ShapeDtypeStruct((ncopies * nh * NP, HID), jnp.float32),
            jax.ShapeDtypeStruct((ncopies * nh * NP,), jnp.float32),
        ),
        mesh=mesh,
        scratch_types=[
            pltpu.VMEM_SHARED((NP, HID), jnp.float32),
            pltpu.VMEM_SHARED((NP,), jnp.float32),
            pltpu.VMEM((NP,), jnp.float32),
            pltpu.VMEM((NP,), jnp.float32),
            pltpu.VMEM((B, HID), jnp.float32),
            pltpu.VMEM((B, HID), jnp.float32),
            pltpu.VMEM((CE,), jnp.int32),
            pltpu.VMEM((CE,), jnp.int32),
            pltpu.VMEM((B,), jnp.int32),
            pltpu.VMEM((B,), jnp.int32),
            pltpu.VMEM((B,), jnp.int32),
            pltpu.VMEM((B,), jnp.int32),
            pltpu.VMEM((B,), jnp.float32),
            pltpu.VMEM((B,), jnp.float32),
            pltpu.SemaphoreType.DMA,
            pltpu.SemaphoreType.DMA,
            pltpu.SemaphoreType.DMA,
            pltpu.SemaphoreType.DMA,
            pltpu.SemaphoreType.DMA,
            pltpu.SemaphoreType.DMA,
        ],
        compiler_params=pltpu.CompilerParams(needs_layout_passes=False),
    )
    def body(feat_hbm, el_hbm, er_hbm, src_hbm, dst_hbm,
             acc_hbm, den_hbm,
             accum_sh, den_sh, el_v, er_v, rows_v0, rows_v1,
             src_c, dst_c, dst_b0, dst_b1, sadj_b0, sadj_b1,
             s_b0, s_b1, sem0, sem1, asem0, asem1, dsem0, dsem1):
        bufs = [(rows_v0, dst_b0, sadj_b0, s_b0, sem0, asem0, dsem0),
                (rows_v1, dst_b1, sadj_b1, s_b1, sem1, asem1, dsem1)]
        c = lax.axis_index("c")
        s = lax.axis_index("s")
        if head_split:
            lane, stride = s, NS
        else:
            lane, stride = c * NS + s, NW
        nq = jnp.where(lane < NCH - (NCH // stride) * stride,
                       NCH // stride + 1, NCH // stride)
        nheads = HPC if head_split else nh

        def head_step(hl, _):
            h = c * HPC + hl if head_split else hl
            pltpu.sync_copy(el_hbm.at[pl.ds(h * NP, NP)], el_v)
            pltpu.sync_copy(er_hbm.at[pl.ds(h * NP, NP)], er_v)

            for k in range(B // 16):
                s_b0[pl.ds(k * 16, 16)] = jnp.zeros((16,), jnp.float32)

            def zrows(j, _):
                for i in range(HID // 16):
                    rows_v0[j, pl.ds(i * 16, 16)] = (
                        jnp.zeros((16,), jnp.float32))
                return _

            lax.fori_loop(0, B, zrows, 0)
            for i in range(RPS // B):
                pltpu.sync_copy(rows_v0, accum_sh.at[pl.ds(s * RPS + i * B, B)])
                pltpu.sync_copy(s_b0, den_sh.at[pl.ds(s * RPS + i * B, B)])
            plsc.subcore_barrier()

            def chunk_step(t, _):
                base = (lane + stride * t) * CE
                pltpu.sync_copy(src_hbm.at[pl.ds(base, CE)], src_c)
                pltpu.sync_copy(dst_hbm.at[pl.ds(base, CE)], dst_c)
                acc_h = [None, None]
                den_h = [None, None]

                def compute(kb):
                    p = kb % 2
                    rows_v, dst_b, sadj_b, s_b, sem, asem, dsem = bufs[p]
                    if acc_h[p] is not None:
                        acc_h[p].wait()
                    if den_h[p] is not None:
                        den_h[p].wait()
                    for k in range(B // 16):
                        o = kb * B + k * 16
                        sv = src_c[pl.ds(o, 16)]
                        dv = dst_c[pl.ds(o, 16)]
                        sadj_b[pl.ds(k * 16, 16)] = sv + h * N
                        dst_b[pl.ds(k * 16, 16)] = dv
                        e = (plsc.load_gather(el_v, [sv])
                             + plsc.load_gather(er_v, [dv]))
                        e = jnp.where(e < 0, e * jnp.float32(0.2), e)
                        s_b[pl.ds(k * 16, 16)] = jnp.exp(e)
                    den_h[p] = pltpu.async_copy(
                        s_b, den_sh.at[dst_b], dsem, add=True)
                    return pltpu.async_copy(feat_hbm.at[sadj_b], rows_v, sem)

                def process(kb, handle):
                    p = kb % 2
                    rows_v, dst_b, sadj_b, s_b, sem, asem, dsem = bufs[p]
                    handle.wait()

                    def row_step(g, _):
                        for r in range(4):
                            j = g * 4 + r
                            sj = plsc.load_gather(
                                s_b, [jnp.full((16,), j, jnp.int32)])
                            for i in range(HID // 16):
                                rows_v[j, pl.ds(i * 16, 16)] = (
                                    rows_v[j, pl.ds(i * 16, 16)] * sj)
                        return _

                    lax.fori_loop(0, B // 4, row_step, 0)
                    acc_h[p] = pltpu.async_copy(
                        rows_v, accum_sh.at[dst_b], asem, add=True)

                handle = compute(0)
                for kb in range(1, CB):
                    nxt = compute(kb)
                    process(kb - 1, handle)
                    handle = nxt
                process(CB - 1, handle)
                for p in range(2):
                    if acc_h[p] is not None:
                        acc_h[p].wait()
                    if den_h[p] is not None:
                        den_h[p].wait()
                return _

            lax.fori_loop(0, nq, chunk_step, 0)
            plsc.subcore_barrier()
            off = (h if head_split else c * nh + h) * NP
            for i in range(RPS // RC):
                r = s * RPS + i * RC
                pltpu.sync_copy(accum_sh.at[pl.ds(r, RC)],
                                acc_hbm.at[pl.ds(off + r, RC)])
            pltpu.sync_copy(den_sh.at[pl.ds(s * RPS, RPS)],
                            den_hbm.at[pl.ds(off + s * RPS, RPS)])
            plsc.subcore_barrier()
            return _

        lax.fori_loop(0, nheads, head_step, 0)

    return body


_sc_pass_l1 = _sc_edge_pass(HEADS, True)
_sc_pass_l2 = _sc_edge_pass(1, False)

R = 200
GRID = N // R


def _prep1_body(x_ref, w1_ref, al_ref, ar_ref, feat_ref, el_ref, er_ref):
    x = x_ref[...]
    els, ers = [], []
    for h in range(HEADS):
        fh = jnp.dot(x, w1_ref[:, pl.ds(h * HID, HID)],
                     preferred_element_type=jnp.float32)
        feat_ref[h] = fh
        els.append(jnp.sum(fh * al_ref[h, :][None, :], axis=1))
        ers.append(jnp.sum(fh * ar_ref[h, :][None, :], axis=1))
    el_ref[...] = jnp.stack(els, axis=1)
    er_ref[...] = jnp.stack(ers, axis=1)


def _prep1(features, W1, al1, ar1):
    return pl.pallas_call(
        _prep1_body,
        grid=(GRID,),
        in_specs=[
            pl.BlockSpec((R, IN), lambda i: (i, 0)),
            pl.BlockSpec((IN, HEADS * HID), lambda i: (0, 0)),
            pl.BlockSpec((HEADS, HID), lambda i: (0, 0)),
            pl.BlockSpec((HEADS, HID), lambda i: (0, 0)),
        ],
        out_specs=[
            pl.BlockSpec((HEADS, R, HID), lambda i: (0, i, 0)),
            pl.BlockSpec((R, HEADS), lambda i: (i, 0)),
            pl.BlockSpec((R, HEADS), lambda i: (i, 0)),
        ],
        out_shape=[
            jax.ShapeDtypeStruct((HEADS, N, HID), jnp.float32),
            jax.ShapeDtypeStruct((N, HEADS), jnp.float32),
            jax.ShapeDtypeStruct((N, HEADS), jnp.float32),
        ],
    )(features, W1, al1, ar1)


def _dense2_body(acc_ref, den_ref, b1_ref, wcat_ref, al_ref, ar_ref,
                 feat2_ref, el_ref, er_ref, res_ref):
    hs = []
    for h in range(HEADS):
        den = den_ref[:, h]
        num = acc_ref[h]
        safe = jnp.where(den > 0, den, jnp.float32(1.0))[:, None]
        hh = jnp.where(den[:, None] > 0, num / safe, jnp.float32(0.0))
        hh = hh + b1_ref[h, :][None, :]
        hh = jnp.where(hh > 0, hh, jnp.exp(jnp.minimum(hh, 0.0)) - 1.0)
        hs.append(hh)
    hcat = jnp.concatenate(hs, axis=1)
    both = jnp.dot(hcat, wcat_ref[...],
                   preferred_element_type=jnp.float32)
    f2 = both[:, :OUT]
    feat2_ref[...] = f2
    el_ref[...] = jnp.sum(f2 * al_ref[0, :][None, :], axis=1)[:, None]
    er_ref[...] = jnp.sum(f2 * ar_ref[0, :][None, :], axis=1)[:, None]
    res_ref[...] = both[:, OUT:]


def _dense2(acc1, den1, b1, Wcat, al2, ar2):
    return pl.pallas_call(
        _dense2_body,
        grid=(GRID,),
        in_specs=[
            pl.BlockSpec((HEADS, R, HID), lambda i: (0, i, 0)),
            pl.BlockSpec((R, HEADS), lambda i: (i, 0)),
            pl.BlockSpec((HEADS, HID), lambda i: (0, 0)),
            pl.BlockSpec((HEADS * HID, 2 * OUT), lambda i: (0, 0)),
            pl.BlockSpec((1, OUT), lambda i: (0, 0)),
            pl.BlockSpec((1, OUT), lambda i: (0, 0)),
        ],
        out_specs=[
            pl.BlockSpec((R, HID), lambda i: (i, 0)),
            pl.BlockSpec((R, 1), lambda i: (i, 0)),
            pl.BlockSpec((R, 1), lambda i: (i, 0)),
            pl.BlockSpec((R, OUT), lambda i: (i, 0)),
        ],
        out_shape=[
            jax.ShapeDtypeStruct((N, HID), jnp.float32),
            jax.ShapeDtypeStruct((NP, 1), jnp.float32),
            jax.ShapeDtypeStruct((NP, 1), jnp.float32),
            jax.ShapeDtypeStruct((N, OUT), jnp.float32),
        ],
    )(acc1, den1, b1, Wcat, al2, ar2)


def _final_body(acc_ref, den_ref, res_ref, b2_ref, out_ref):
    comb = acc_ref[0] + acc_ref[1]
    den = den_ref[:, 0] + den_ref[:, 1]
    safe = jnp.where(den > 0, den, jnp.float32(1.0))[:, None]
    rst = jnp.where(den[:, None] > 0, comb / safe, jnp.float32(0.0))
    out_ref[...] = rst + res_ref[...] + b2_ref[0, :][None, :]


def _final(acc2, den2, res, b2):
    return pl.pallas_call(
        _final_body,
        grid=(GRID,),
        in_specs=[
            pl.BlockSpec((NC, R, HID), lambda i: (0, i, 0)),
            pl.BlockSpec((R, NC), lambda i: (i, 0)),
            pl.BlockSpec((R, OUT), lambda i: (i, 0)),
            pl.BlockSpec((1, OUT), lambda i: (0, 0)),
        ],
        out_specs=pl.BlockSpec((R, OUT), lambda i: (i, 0)),
        out_shape=jax.ShapeDtypeStruct((N, OUT), jnp.float32),
    )(acc2, den2, res, b2)


def _pad_nodes(x):
    k = x.shape[1]
    return jnp.pad(x.T, ((0, 0), (0, NP - N))).reshape(k * NP)


def kernel(graph, features, W1, al1, ar1, b1, W2, al2, ar2, b2, Wres2):
    src = graph[0].astype(jnp.int32)
    dst = graph[1].astype(jnp.int32)

    feat1, el1, er1 = _prep1(features, W1, al1, ar1)
    acc1, den1 = _sc_pass_l1(feat1.reshape(HEADS * N, HID),
                             _pad_nodes(el1), _pad_nodes(er1), src, dst)
    Wcat = jnp.concatenate([W2, Wres2], axis=1)
    feat2, el2, er2, res = _dense2(acc1.reshape(HEADS, NP, HID),
                                   den1.reshape(HEADS, NP).T,
                                   b1, Wcat, al2, ar2)
    acc2, den2 = _sc_pass_l2(feat2, el2.reshape(-1), er2.reshape(-1),
                             src, dst)
    return _final(acc2.reshape(NC, NP, HID), den2.reshape(NC, NP).T,
                  res, b2)


# speedup vs baseline: 23.3508x; 1.0014x over previous
import functools

import jax
import jax.numpy as jnp
from jax import lax
from jax.experimental import pallas as pl
from jax.experimental.pallas import tpu as pltpu
from jax.experimental.pallas import tpu_sc as plsc

N = 10000
E = 320000
IN = 128
HID = 128
HEADS = 8
OUT = 128

NC = 2
NS = 16
NW = NC * NS
B = 64
CB = 10
CE = CB * B
NCH = E // CE
NP = 10240
RPS = NP // NS
RC = 128
HPC = HEADS // NC


def _sc_edge_pass(nh, head_split):
    ncopies = 1 if head_split else NC
    mesh = plsc.VectorSubcoreMesh(core_axis_name="c", subcore_axis_name="s",
                                  num_cores=NC, num_subcores=NS)

    @functools.partial(
        pl.kernel,
        out_type=(
            jax.ShapeDtypeStruct((ncopies * nh * NP, HID), jnp.float32),
            jax.ShapeDtypeStruct((ncopies * nh * NP,), jnp.float32),
        ),
        mesh=mesh,
        scratch_types=[
            pltpu.VMEM_SHARED((NP, HID), jnp.float32),
            pltpu.VMEM_SHARED((NP,), jnp.float32),
            pltpu.VMEM((NP,), jnp.float32),
            pltpu.VMEM((NP,), jnp.float32),
            pltpu.VMEM((B, HID), jnp.float32),
            pltpu.VMEM((B, HID), jnp.float32),
            pltpu.VMEM((CE,), jnp.int32),
            pltpu.VMEM((CE,), jnp.int32),
            pltpu.VMEM((B,), jnp.int32),
            pltpu.VMEM((B,), jnp.int32),
            pltpu.VMEM((B,), jnp.int32),
            pltpu.VMEM((B,), jnp.int32),
            pltpu.VMEM((B,), jnp.float32),
            pltpu.VMEM((B,), jnp.float32),
            pltpu.SemaphoreType.DMA,
            pltpu.SemaphoreType.DMA,
            pltpu.SemaphoreType.DMA,
            pltpu.SemaphoreType.DMA,
            pltpu.SemaphoreType.DMA,
            pltpu.SemaphoreType.DMA,
        ],
        compiler_params=pltpu.CompilerParams(needs_layout_passes=False),
    )
    def body(feat_hbm, el_hbm, er_hbm, src_hbm, dst_hbm,
             acc_hbm, den_hbm,
             accum_sh, den_sh, el_v, er_v, rows_v0, rows_v1,
             src_c, dst_c, dst_b0, dst_b1, sadj_b0, sadj_b1,
             s_b0, s_b1, sem0, sem1, asem0, asem1, dsem0, dsem1):
        bufs = [(rows_v0, dst_b0, sadj_b0, s_b0, sem0, asem0, dsem0),
                (rows_v1, dst_b1, sadj_b1, s_b1, sem1, asem1, dsem1)]
        c = lax.axis_index("c")
        s = lax.axis_index("s")
        if head_split:
            lane, stride = s, NS
        else:
            lane, stride = c * NS + s, NW
        nq = jnp.where(lane < NCH - (NCH // stride) * stride,
                       NCH // stride + 1, NCH // stride)
        nheads = HPC if head_split else nh

        def head_step(hl, _):
            h = c * HPC + hl if head_split else hl
            pltpu.sync_copy(el_hbm.at[pl.ds(h * NP, NP)], el_v)
            pltpu.sync_copy(er_hbm.at[pl.ds(h * NP, NP)], er_v)

            for k in range(B // 16):
                s_b0[pl.ds(k * 16, 16)] = jnp.zeros((16,), jnp.float32)

            def zrows(j, _):
                for i in range(HID // 16):
                    rows_v0[j, pl.ds(i * 16, 16)] = (
                        jnp.zeros((16,), jnp.float32))
                return _

            lax.fori_loop(0, B, zrows, 0)
            for i in range(RPS // B):
                pltpu.sync_copy(rows_v0, accum_sh.at[pl.ds(s * RPS + i * B, B)])
                pltpu.sync_copy(s_b0, den_sh.at[pl.ds(s * RPS + i * B, B)])
            plsc.subcore_barrier()

            def chunk_step(t, _):
                base = (lane + stride * t) * CE
                pltpu.sync_copy(src_hbm.at[pl.ds(base, CE)], src_c)
                pltpu.sync_copy(dst_hbm.at[pl.ds(base, CE)], dst_c)
                acc_h = [None, None]
                den_h = [None, None]

                def compute(kb):
                    p = kb % 2
                    rows_v, dst_b, sadj_b, s_b, sem, asem, dsem = bufs[p]
                    if acc_h[p] is not None:
                        acc_h[p].wait()
                    if den_h[p] is not None:
                        den_h[p].wait()
                    for k in range(B // 16):
                        o = kb * B + k * 16
                        sv = src_c[pl.ds(o, 16)]
                        dv = dst_c[pl.ds(o, 16)]
                        sadj_b[pl.ds(k * 16, 16)] = sv + h * N
                        dst_b[pl.ds(k * 16, 16)] = dv
                        e = (plsc.load_gather(el_v, [sv])
                             + plsc.load_gather(er_v, [dv]))
                        e = jnp.where(e < 0, e * jnp.float32(0.2), e)
                        s_b[pl.ds(k * 16, 16)] = jnp.exp(e)
                    den_h[p] = pltpu.async_copy(
                        s_b, den_sh.at[dst_b], dsem, add=True)
                    return pltpu.async_copy(feat_hbm.at[sadj_b], rows_v, sem)

                def process(kb, handle):
                    p = kb % 2
                    rows_v, dst_b, sadj_b, s_b, sem, asem, dsem = bufs[p]
                    handle.wait()

                    def row_step(g, _):
                        for r in range(8):
                            j = g * 8 + r
                            sj = plsc.load_gather(
                                s_b, [jnp.full((16,), j, jnp.int32)])
                            for i in range(HID // 16):
                                rows_v[j, pl.ds(i * 16, 16)] = (
                                    rows_v[j, pl.ds(i * 16, 16)] * sj)
                        return _

                    lax.fori_loop(0, B // 8, row_step, 0)
                    acc_h[p] = pltpu.async_copy(
                        rows_v, accum_sh.at[dst_b], asem, add=True)

                handle = compute(0)
                for kb in range(1, CB):
                    nxt = compute(kb)
                    process(kb - 1, handle)
                    handle = nxt
                process(CB - 1, handle)
                for p in range(2):
                    if acc_h[p] is not None:
                        acc_h[p].wait()
                    if den_h[p] is not None:
                        den_h[p].wait()
                return _

            lax.fori_loop(0, nq, chunk_step, 0)
            plsc.subcore_barrier()
            off = (h if head_split else c * nh + h) * NP
            for i in range(RPS // RC):
                r = s * RPS + i * RC
                pltpu.sync_copy(accum_sh.at[pl.ds(r, RC)],
                                acc_hbm.at[pl.ds(off + r, RC)])
            pltpu.sync_copy(den_sh.at[pl.ds(s * RPS, RPS)],
                            den_hbm.at[pl.ds(off + s * RPS, RPS)])
            plsc.subcore_barrier()
            return _

        lax.fori_loop(0, nheads, head_step, 0)

    return body


_sc_pass_l1 = _sc_edge_pass(HEADS, True)
_sc_pass_l2 = _sc_edge_pass(1, False)

R = 200
GRID = N // R


def _prep1_body(x_ref, w1_ref, al_ref, ar_ref, feat_ref, el_ref, er_ref):
    x = x_ref[...]
    els, ers = [], []
    for h in range(HEADS):
        fh = jnp.dot(x, w1_ref[:, pl.ds(h * HID, HID)],
                     preferred_element_type=jnp.float32)
        feat_ref[h] = fh
        els.append(jnp.sum(fh * al_ref[h, :][None, :], axis=1))
        ers.append(jnp.sum(fh * ar_ref[h, :][None, :], axis=1))
    el_ref[...] = jnp.stack(els, axis=1)
    er_ref[...] = jnp.stack(ers, axis=1)


def _prep1(features, W1, al1, ar1):
    return pl.pallas_call(
        _prep1_body,
        grid=(GRID,),
        in_specs=[
            pl.BlockSpec((R, IN), lambda i: (i, 0)),
            pl.BlockSpec((IN, HEADS * HID), lambda i: (0, 0)),
            pl.BlockSpec((HEADS, HID), lambda i: (0, 0)),
            pl.BlockSpec((HEADS, HID), lambda i: (0, 0)),
        ],
        out_specs=[
            pl.BlockSpec((HEADS, R, HID), lambda i: (0, i, 0)),
            pl.BlockSpec((R, HEADS), lambda i: (i, 0)),
            pl.BlockSpec((R, HEADS), lambda i: (i, 0)),
        ],
        out_shape=[
            jax.ShapeDtypeStruct((HEADS, N, HID), jnp.float32),
            jax.ShapeDtypeStruct((N, HEADS), jnp.float32),
            jax.ShapeDtypeStruct((N, HEADS), jnp.float32),
        ],
    )(features, W1, al1, ar1)


def _dense2_body(acc_ref, den_ref, b1_ref, wcat_ref, al_ref, ar_ref,
                 feat2_ref, el_ref, er_ref, res_ref):
    hs = []
    for h in range(HEADS):
        den = den_ref[:, h]
        num = acc_ref[h]
        safe = jnp.where(den > 0, den, jnp.float32(1.0))[:, None]
        hh = jnp.where(den[:, None] > 0, num / safe, jnp.float32(0.0))
        hh = hh + b1_ref[h, :][None, :]
        hh = jnp.where(hh > 0, hh, jnp.exp(jnp.minimum(hh, 0.0)) - 1.0)
        hs.append(hh)
    hcat = jnp.concatenate(hs, axis=1)
    both = jnp.dot(hcat, wcat_ref[...],
                   preferred_element_type=jnp.float32)
    f2 = both[:, :OUT]
    feat2_ref[...] = f2
    el_ref[...] = jnp.sum(f2 * al_ref[0, :][None, :], axis=1)[:, None]
    er_ref[...] = jnp.sum(f2 * ar_ref[0, :][None, :], axis=1)[:, None]
    res_ref[...] = both[:, OUT:]


def _dense2(acc1, den1, b1, Wcat, al2, ar2):
    return pl.pallas_call(
        _dense2_body,
        grid=(GRID,),
        in_specs=[
            pl.BlockSpec((HEADS, R, HID), lambda i: (0, i, 0)),
            pl.BlockSpec((R, HEADS), lambda i: (i, 0)),
            pl.BlockSpec((HEADS, HID), lambda i: (0, 0)),
            pl.BlockSpec((HEADS * HID, 2 * OUT), lambda i: (0, 0)),
            pl.BlockSpec((1, OUT), lambda i: (0, 0)),
            pl.BlockSpec((1, OUT), lambda i: (0, 0)),
        ],
        out_specs=[
            pl.BlockSpec((R, HID), lambda i: (i, 0)),
            pl.BlockSpec((R, 1), lambda i: (i, 0)),
            pl.BlockSpec((R, 1), lambda i: (i, 0)),
            pl.BlockSpec((R, OUT), lambda i: (i, 0)),
        ],
        out_shape=[
            jax.ShapeDtypeStruct((N, HID), jnp.float32),
            jax.ShapeDtypeStruct((NP, 1), jnp.float32),
            jax.ShapeDtypeStruct((NP, 1), jnp.float32),
            jax.ShapeDtypeStruct((N, OUT), jnp.float32),
        ],
    )(acc1, den1, b1, Wcat, al2, ar2)


def _final_body(acc_ref, den_ref, res_ref, b2_ref, out_ref):
    comb = acc_ref[0] + acc_ref[1]
    den = den_ref[:, 0] + den_ref[:, 1]
    safe = jnp.where(den > 0, den, jnp.float32(1.0))[:, None]
    rst = jnp.where(den[:, None] > 0, comb / safe, jnp.float32(0.0))
    out_ref[...] = rst + res_ref[...] + b2_ref[0, :][None, :]


def _final(acc2, den2, res, b2):
    return pl.pallas_call(
        _final_body,
        grid=(GRID,),
        in_specs=[
            pl.BlockSpec((NC, R, HID), lambda i: (0, i, 0)),
            pl.BlockSpec((R, NC), lambda i: (i, 0)),
            pl.BlockSpec((R, OUT), lambda i: (i, 0)),
            pl.BlockSpec((1, OUT), lambda i: (0, 0)),
        ],
        out_specs=pl.BlockSpec((R, OUT), lambda i: (i, 0)),
        out_shape=jax.ShapeDtypeStruct((N, OUT), jnp.float32),
    )(acc2, den2, res, b2)


def _pad_nodes(x):
    k = x.shape[1]
    return jnp.pad(x.T, ((0, 0), (0, NP - N))).reshape(k * NP)


def kernel(graph, features, W1, al1, ar1, b1, W2, al2, ar2, b2, Wres2):
    src = graph[0].astype(jnp.int32)
    dst = graph[1].astype(jnp.int32)

    feat1, el1, er1 = _prep1(features, W1, al1, ar1)
    acc1, den1 = _sc_pass_l1(feat1.reshape(HEADS * N, HID),
                             _pad_nodes(el1), _pad_nodes(er1), src, dst)
    Wcat = jnp.concatenate([W2, Wres2], axis=1)
    feat2, el2, er2, res = _dense2(acc1.reshape(HEADS, NP, HID),
                                   den1.reshape(HEADS, NP).T,
                                   b1, Wcat, al2, ar2)
    acc2, den2 = _sc_pass_l2(feat2, el2.reshape(-1), er2.reshape(-1),
                             src, dst)
    return _final(acc2.reshape(NC, NP, HID), den2.reshape(NC, NP).T,
                  res, b2)
